# baseline (speedup 1.0000x reference)
import jax
import jax.numpy as jnp
from jax.experimental import pallas as pl


def _hgat_layer(x, edge_index, node_type, edge_attr, edge_type, W, a_src, a_dst, We, nte, ete, b, Wres, concat):
    n = x.shape[0]
    H, C = a_src.shape
    xin = x + nte[node_type]
    h = (xin @ W).reshape(n, H, C)
    asrc = (h * a_src[None, :, :]).sum(-1)
    adst = (h * a_dst[None, :, :]).sum(-1)
    ef = edge_attr + ete[edge_type]
    ae = ef @ We
    src = edge_index[0]
    dst = edge_index[1]
    logits = jax.nn.leaky_relu(asrc[src] + adst[dst] + ae, 0.2)
    m = jax.ops.segment_max(logits, dst, num_segments=n)
    m = jnp.where(jnp.isfinite(m), m, 0.0)
    ea = jnp.exp(logits - m[dst])
    den = jax.ops.segment_sum(ea, dst, num_segments=n)
    alpha = ea / (den[dst] + 1e-16)
    msg = h[src] * alpha[:, :, None]
    out = jax.ops.segment_sum(msg, dst, num_segments=n)
    out = out.reshape(n, H * C) if concat else out.mean(axis=1)
    res = x if Wres is None else x @ Wres
    return out, res, b


def _final_add_kernel(o_ref, r_ref, b_ref, out_ref):
    out_ref[...] = o_ref[...] + r_ref[...] + b_ref[...]


def _final_add(out, res, b):
    n, d = out.shape
    blk = 1000
    return pl.pallas_call(
        _final_add_kernel,
        grid=(n // blk,),
        in_specs=[
            pl.BlockSpec((blk, d), lambda i: (i, 0)),
            pl.BlockSpec((blk, d), lambda i: (i, 0)),
            pl.BlockSpec((1, d), lambda i: (0, 0)),
        ],
        out_specs=pl.BlockSpec((blk, d), lambda i: (i, 0)),
        out_shape=jax.ShapeDtypeStruct((n, d), out.dtype),
    )(out, res, b.reshape(1, d))


def kernel(x, edge_index, node_type, edge_attr, edge_type, W1, att_src1, att_dst1, Wedge1, ntype1, etype1, Wres1, b1, gamma, beta, W2, att_src2, att_dst2, Wedge2, ntype2, etype2, b2):
    o, r, b = _hgat_layer(x, edge_index, node_type, edge_attr, edge_type, W1, att_src1, att_dst1, Wedge1, ntype1, etype1, b1, Wres1, True)
    h = _final_add(o, r, b)
    mu = h.mean(axis=0)
    var = h.var(axis=0)
    h = (h - mu) / jnp.sqrt(var + 1e-5) * gamma + beta
    o, r, b = _hgat_layer(h, edge_index, node_type, edge_attr, edge_type, W2, att_src2, att_dst2, Wedge2, ntype2, etype2, b2, None, False)
    return _final_add(o, r, b)


# speedup vs baseline: 8.6358x; 8.6358x over previous
import functools

import jax
import jax.numpy as jnp
from jax import lax
from jax.experimental import pallas as pl
from jax.experimental.pallas import tpu as pltpu
from jax.experimental.pallas import tpu_sc as plsc

N = 10000
E = 160000
HID = 128
HEADS = 4
D1 = 512
NT = 4
ET = 8
EPT = E // 32
EPT4 = EPT * 4
NG = EPT // 16
GEOM = {128: dict(CHUNK=1024, NCH=10), 512: dict(CHUNK=256, NCH=40)}
PREG = EPT + 16 * 48 + 40
OP = 48


def _node_pre_kernel(x_ref, nt_ref, ntab_ref, vsd_ref, wres_ref, xin_ref, asd_ref, res_ref):
    x = x_ref[...]
    nt = nt_ref[...]
    emb = jnp.zeros_like(x)
    for t in range(NT):
        emb = jnp.where(nt == t, jnp.broadcast_to(ntab_ref[t:t + 1, :], x.shape), emb)
    xin = x + emb
    xin_ref[...] = xin
    asd_ref[...] = jnp.dot(xin, vsd_ref[...], preferred_element_type=jnp.float32)
    res_ref[...] = jnp.dot(x, wres_ref[...], preferred_element_type=jnp.float32)


def _node_pre(x, nt2d, ntab, vsd, wres):
    n, d = x.shape
    blk = 1000
    return pl.pallas_call(
        _node_pre_kernel,
        grid=(n // blk,),
        in_specs=[
            pl.BlockSpec((blk, d), lambda i: (i, 0)),
            pl.BlockSpec((blk, 1), lambda i: (i, 0)),
            pl.BlockSpec(ntab.shape, lambda i: (0, 0)),
            pl.BlockSpec(vsd.shape, lambda i: (0, 0)),
            pl.BlockSpec(wres.shape, lambda i: (0, 0)),
        ],
        out_specs=[
            pl.BlockSpec((blk, d), lambda i: (i, 0)),
            pl.BlockSpec((blk, 8), lambda i: (i, 0)),
            pl.BlockSpec((blk, wres.shape[1]), lambda i: (i, 0)),
        ],
        out_shape=[
            jax.ShapeDtypeStruct((n, d), jnp.float32),
            jax.ShapeDtypeStruct((n, 8), jnp.float32),
            jax.ShapeDtypeStruct((n, wres.shape[1]), jnp.float32),
        ],
    )(x, nt2d, ntab, vsd, wres)


def _node_pre2_kernel(h_ref, st_ref, nt_ref, ntab_ref, vsd_ref, bn_ref, xin_ref, asd_ref):
    h = h_ref[...]
    nt = nt_ref[...]
    s = st_ref[0:1, :]
    t = st_ref[1:2, :]
    bn = h * s + t
    emb = jnp.zeros_like(h)
    for tt in range(NT):
        emb = jnp.where(nt == tt, jnp.broadcast_to(ntab_ref[tt:tt + 1, :], h.shape), emb)
    xin = bn + emb
    bn_ref[...] = bn
    xin_ref[...] = xin
    asd_ref[...] = jnp.dot(xin, vsd_ref[...], preferred_element_type=jnp.float32)


def _node_pre2(h, st, nt2d, ntab, vsd):
    n, d = h.shape
    blk = 1000
    return pl.pallas_call(
        _node_pre2_kernel,
        grid=(n // blk,),
        in_specs=[
            pl.BlockSpec((blk, d), lambda i: (i, 0)),
            pl.BlockSpec(st.shape, lambda i: (0, 0)),
            pl.BlockSpec((blk, 1), lambda i: (i, 0)),
            pl.BlockSpec(ntab.shape, lambda i: (0, 0)),
            pl.BlockSpec(vsd.shape, lambda i: (0, 0)),
        ],
        out_specs=[
            pl.BlockSpec((blk, d), lambda i: (i, 0)),
            pl.BlockSpec((blk, d), lambda i: (i, 0)),
            pl.BlockSpec((blk, 8), lambda i: (i, 0)),
        ],
        out_shape=[
            jax.ShapeDtypeStruct((n, d), jnp.float32),
            jax.ShapeDtypeStruct((n, d), jnp.float32),
            jax.ShapeDtypeStruct((n, 8), jnp.float32),
        ],
    )(h, st, nt2d, ntab, vsd)


def _edge_pre_kernel(ea_ref, et_ref, etab_ref, we_ref, ae_ref):
    et = et_ref[...]
    acc = jnp.zeros((ea_ref.shape[0], 8), jnp.float32)
    for t in range(ET):
        acc = jnp.where(et == t, jnp.broadcast_to(etab_ref[t:t + 1, :], acc.shape), acc)
    ae_ref[...] = jnp.dot(ea_ref[...], we_ref[...], preferred_element_type=jnp.float32) + acc


def _edge_pre(edge_attr, et2d, etab, we):
    e, d = edge_attr.shape
    blk = 2000
    return pl.pallas_call(
        _edge_pre_kernel,
        grid=(e // blk,),
        in_specs=[
            pl.BlockSpec((blk, d), lambda i: (i, 0)),
            pl.BlockSpec((blk, 1), lambda i: (i, 0)),
            pl.BlockSpec(etab.shape, lambda i: (0, 0)),
            pl.BlockSpec(we.shape, lambda i: (0, 0)),
        ],
        out_specs=pl.BlockSpec((blk, 8), lambda i: (i, 0)),
        out_shape=jax.ShapeDtypeStruct((e, 8), jnp.float32),
    )(edge_attr, et2d, etab, we)


def _counts_kernel(csh, dst_ref, cnt_ref):
    cid = dst_ref[...] >> csh
    cidr = cid.reshape(1, EPT, 1)
    ids = lax.broadcasted_iota(jnp.int32, (1, 1, OP), 2)
    eq = jnp.where(cidr == ids, 1, 0)
    cnt_ref[...] = jnp.sum(eq, axis=1, keepdims=True)


def _counts(dst3, chunk):
    csh = chunk.bit_length() - 1
    return pl.pallas_call(
        functools.partial(_counts_kernel, csh),
        grid=(32,),
        in_specs=[pl.BlockSpec((1, 1, EPT), lambda i: (i, 0, 0))],
        out_specs=pl.BlockSpec((1, 1, OP), lambda i: (i, 0, 0)),
        out_shape=jax.ShapeDtypeStruct((32, 1, OP), jnp.int32),
    )(dst3)


def _combine1_kernel(aggA_ref, aggB_ref, denA_ref, denB_ref, bd_ref, res_ref, b_ref,
                     out_ref, stats_ref):
    i = pl.program_id(0)
    agg = aggA_ref[...] + aggB_ref[...]
    den = denA_ref[...] + denB_ref[...] + 1e-16
    blk = agg.shape[0]
    d = agg.shape[1] // HEADS
    deni = (1.0 / den).reshape(blk, HEADS, 1)
    norm = (agg.reshape(blk, HEADS, d) * deni).reshape(blk, HEADS * d)
    out = jnp.dot(norm, bd_ref[...], preferred_element_type=jnp.float32)
    out = out + res_ref[...] + b_ref[...]
    out_ref[...] = out
    s = jnp.sum(out, axis=0, keepdims=True)
    ss = jnp.sum(out * out, axis=0, keepdims=True)
    st = jnp.concatenate([s, ss, jnp.zeros((6, out.shape[1]), jnp.float32)], axis=0)

    @pl.when(i == 0)
    def _():
        stats_ref[...] = jnp.zeros_like(stats_ref)

    stats_ref[...] += st


def _combine1(aggA, aggB, denA, denB, bd, res, b2d):
    n, dcat = aggA.shape
    blk = 1000
    return pl.pallas_call(
        _combine1_kernel,
        grid=(n // blk,),
        in_specs=[
            pl.BlockSpec((blk, dcat), lambda i: (i, 0)),
            pl.BlockSpec((blk, dcat), lambda i: (i, 0)),
            pl.BlockSpec((blk, HEADS), lambda i: (i, 0)),
            pl.BlockSpec((blk, HEADS), lambda i: (i, 0)),
            pl.BlockSpec(bd.shape, lambda i: (0, 0)),
            pl.BlockSpec((blk, bd.shape[1]), lambda i: (i, 0)),
            pl.BlockSpec(b2d.shape, lambda i: (0, 0)),
        ],
        out_specs=[
            pl.BlockSpec((blk, bd.shape[1]), lambda i: (i, 0)),
            pl.BlockSpec((8, bd.shape[1]), lambda i: (0, 0)),
        ],
        out_shape=[
            jax.ShapeDtypeStruct((n, bd.shape[1]), jnp.float32),
            jax.ShapeDtypeStruct((8, bd.shape[1]), jnp.float32),
        ],
    )(aggA, aggB, denA, denB, bd, res, b2d)


def _combine2_kernel(aggA_ref, aggB_ref, denA_ref, denB_ref, wm_ref, res_ref, b_ref, out_ref):
    agg = aggA_ref[...] + aggB_ref[...]
    den = denA_ref[...] + denB_ref[...] + 1e-16
    blk = agg.shape[0]
    d = agg.shape[1] // HEADS
    deni = (1.0 / den).reshape(blk, HEADS, 1)
    norm = (agg.reshape(blk, HEADS, d) * deni).reshape(blk, HEADS * d)
    out = jnp.dot(norm, wm_ref[...], preferred_element_type=jnp.float32)
    out_ref[...] = out + res_ref[...] + b_ref[...]


def _combine2(aggA, aggB, denA, denB, wm, res, b2d):
    n, dcat = aggA.shape
    blk = 1000
    return pl.pallas_call(
        _combine2_kernel,
        grid=(n // blk,),
        in_specs=[
            pl.BlockSpec((blk, dcat), lambda i: (i, 0)),
            pl.BlockSpec((blk, dcat), lambda i: (i, 0)),
            pl.BlockSpec((blk, HEADS), lambda i: (i, 0)),
            pl.BlockSpec((blk, HEADS), lambda i: (i, 0)),
            pl.BlockSpec(wm.shape, lambda i: (0, 0)),
            pl.BlockSpec((blk, wm.shape[1]), lambda i: (i, 0)),
            pl.BlockSpec(b2d.shape, lambda i: (0, 0)),
        ],
        out_specs=pl.BlockSpec((blk, wm.shape[1]), lambda i: (i, 0)),
        out_shape=jax.ShapeDtypeStruct((n, wm.shape[1]), jnp.float32),
    )(aggA, aggB, denA, denB, wm, res, b2d)


def _vg(v, idx):
    dn = lax.GatherDimensionNumbers(
        offset_dims=(), collapsed_slice_dims=(0,), start_index_map=(0,))
    return lax.gather(v, idx[:, None], dn, (1,),
                      mode=lax.GatherScatterMode.PROMISE_IN_BOUNDS)


def _make_sc_edge(D):
    geom = GEOM[D]
    CHUNK, NCH = geom["CHUNK"], geom["NCH"]
    CSH = CHUNK.bit_length() - 1
    NCV = (NCH + 15) // 16
    SEGS = D // 128
    NSC = (64 * SEGS + 127) // 128
    RPB = 64 * SEGS // NSC
    EPB = 16 // NSC
    ROWS = CHUNK * 4 * SEGS
    RZ = ROWS // 16

    mesh = plsc.VectorSubcoreMesh(core_axis_name="c", subcore_axis_name="s")

    @functools.partial(
        pl.kernel,
        mesh=mesh,
        out_type=[jax.ShapeDtypeStruct((2, NCH, ROWS, 128), jnp.float32),
                  jax.ShapeDtypeStruct((32 * EPT4,), jnp.float32),
                  jax.ShapeDtypeStruct((2, NCH, CHUNK * 4), jnp.float32)],
        scratch_types=[
            pltpu.VMEM((EPT + 64,), jnp.int32),
            pltpu.VMEM((EPT + 64,), jnp.int32),
            pltpu.VMEM((EPT4 + 64,), jnp.float32),
            pltpu.VMEM((PREG,), jnp.int32),
            pltpu.VMEM((16, D), jnp.float32),
            pltpu.VMEM((64,), jnp.float32),
            pltpu.VMEM((64,), jnp.int32),
        ] + [pltpu.VMEM((RPB, 128), jnp.float32) for _ in range(NSC)]
          + [pltpu.VMEM((RPB,), jnp.int32) for _ in range(NSC)] + [
            pltpu.VMEM((64,), jnp.int32),
            pltpu.VMEM((64,), jnp.float32),
            pltpu.VMEM((64,), jnp.float32),
            pltpu.VMEM((64,), jnp.float32),
            pltpu.VMEM((64,), jnp.int32),
            pltpu.VMEM((64,), jnp.int32),
            pltpu.VMEM((16,), jnp.int32),
            pltpu.VMEM((16,), jnp.int32),
            pltpu.VMEM((OP,), jnp.int32),
            pltpu.SMEM((OP,), jnp.int32),
            pltpu.VMEM_SHARED((32 * OP,), jnp.int32),
            pltpu.VMEM_SHARED((16 * PREG,), jnp.int32),
            pltpu.VMEM_SHARED((N * 4,), jnp.float32),
            pltpu.VMEM_SHARED((N * 4,), jnp.float32),
            pltpu.VMEM_SHARED((ROWS, 128), jnp.float32),
            pltpu.VMEM_SHARED((CHUNK * 4 + 16,), jnp.float32),
        ],
    )
    def sc_edge(src_hbm, dst_hbm, aef_hbm, asrcf_hbm, adstf_hbm, xin_hbm,
                zeros_hbm, zerod_hbm, offb_hbm,
                out_hbm, eaout_hbm, den_hbm,
                src_v, dst_v, ea_v, pos_v, gbuf, easc, sidx, *rest):
        sbufs = rest[:NSC]
        sidxs = rest[NSC:2 * NSC]
        (eaidx, ea_tmp, atmp, btmp, cposbuf, cvalbuf, srcbuf, dstbuf,
         offb_v, offb_smem, offs_sh, pos_sp, asrc_sh, adst_sh, agg_sh,
         den_sh) = rest[2 * NSC:]
        cc = lax.axis_index("c")
        sid = lax.axis_index("s")
        wid = sid * 2 + cc
        iota = lax.iota(jnp.int32, 16)
        lane14 = iota >> 2
        hpat = iota & 3
        zi = jnp.zeros((16,), jnp.int32)
        zf = jnp.zeros((16,), jnp.float32)
        padpos = jnp.broadcast_to(EPT, (16,))
        nvec = jnp.broadcast_to(N, (16,))

        base = wid * EPT
        pltpu.sync_copy(src_hbm.at[pl.ds(base, EPT)], src_v.at[pl.ds(0, EPT)])
        pltpu.sync_copy(dst_hbm.at[pl.ds(base, EPT)], dst_v.at[pl.ds(0, EPT)])
        pltpu.sync_copy(aef_hbm.at[pl.ds(base * 4, EPT4)],
                        ea_v.at[pl.ds(0, EPT4)])
        for q in range(4):
            src_v[pl.ds(EPT + q * 16, 16)] = zi
            dst_v[pl.ds(EPT + q * 16, 16)] = nvec

        @pl.when(sid == 0)
        def _():
            pltpu.sync_copy(asrcf_hbm, asrc_sh)
            pltpu.sync_copy(adstf_hbm, adst_sh)
            pltpu.sync_copy(offb_hbm, offs_sh)

        plsc.subcore_barrier()
        pltpu.sync_copy(offs_sh.at[pl.ds(wid * OP, OP)], offb_v)
        pltpu.sync_copy(offs_sh.at[pl.ds(wid * OP, OP)], offb_smem)

        def ea_body(i, carry):
            srcv = src_v[pl.ds(i * 16, 16)]
            dstv = dst_v[pl.ds(i * 16, 16)]
            for j in range(4):
                pat = lane14 + 4 * j
                srep = _vg(srcv, pat)
                drep = _vg(dstv, pat)
                drep = jnp.where(drep > N - 1, zi, drep)
                eaidx[pl.ds(j * 16, 16)] = srep * 4 + hpat
                sidx[pl.ds(j * 16, 16)] = drep * 4 + hpat
            pltpu.sync_copy(asrc_sh.at[eaidx], atmp)
            pltpu.sync_copy(adst_sh.at[sidx], btmp)
            for j in range(4):
                v = atmp[pl.ds(j * 16, 16)] + btmp[pl.ds(j * 16, 16)] \
                    + ea_v[pl.ds(i * 64 + j * 16, 16)]
                v = jnp.where(v >= 0.0, v, 0.2 * v)
                ea_v[pl.ds(i * 64 + j * 16, 16)] = jnp.exp(v)
            return carry

        lax.fori_loop(0, NG + 1, ea_body, 0)
        for q in range(4):
            ea_v[pl.ds(EPT4 + q * 16, 16)] = zf
        pltpu.sync_copy(ea_v.at[pl.ds(0, EPT4)],
                        eaout_hbm.at[pl.ds(wid * EPT4, EPT4)])

        def pf_body(i, carry):
            pos_v[pl.ds(i * 16, 16)] = padpos
            return carry

        lax.fori_loop(0, PREG // 16, pf_body, 0)
        pltpu.sync_copy(pos_v, pos_sp.at[pl.ds(sid * PREG, PREG)])

        def comp_body(g, counters):
            cnts = list(counters)
            for j4 in range(4):
                i = g * 4 + j4
                dstv = dst_v[pl.ds(i * 16, 16)]
                posv = i * 16 + iota
                okm = posv < EPT
                cid = jnp.where(okm, dstv >> CSH, iota * 0 + 99)
                rank = zi
                hists = [zi] * NCV
                for l in range(16):
                    cl = _vg(cid, jnp.broadcast_to(l, (16,)))
                    rank = rank + jnp.where((cid == cl) & (iota > l), 1, 0)
                    for k in range(NCV):
                        hists[k] = hists[k] + jnp.where(cl - 16 * k == iota, 1, 0)
                slot = zi
                for k in range(NCV):
                    ck = cid - 16 * k
                    ck = jnp.where(ck < 0, zi, ck)
                    ck = jnp.where(ck > 15, zi, ck)
                    bask = _vg(cnts[k] + offb_v[pl.ds(16 * k, 16)] * 16, ck)
                    if k == 0:
                        slot = bask
                    else:
                        slot = jnp.where(cid > 16 * k - 1, bask, slot)
                slot = slot + rank
                slot = jnp.where(okm, slot, PREG - 16 + (iota & 7))
                cposbuf[pl.ds(j4 * 16, 16)] = slot + sid * PREG
                cvalbuf[pl.ds(j4 * 16, 16)] = posv
                for k in range(NCV):
                    cnts[k] = cnts[k] + hists[k]
            pltpu.sync_copy(cvalbuf, pos_sp.at[cposbuf])
            return tuple(cnts)

        lax.fori_loop(0, (NG + 1 + 3) // 4, comp_body, tuple([zi] * NCV))
        pltpu.sync_copy(pos_sp.at[pl.ds(sid * PREG, PREG)], pos_v)

        def chunk_body(ch, carry):
            lo = ch * CHUNK
            hi = lo + CHUNK
            pltpu.sync_copy(zeros_hbm, agg_sh.at[pl.ds(sid * RZ, RZ), :])

            @pl.when(sid == 0)
            def _():
                pltpu.sync_copy(zerod_hbm, den_sh.at[pl.ds(0, CHUNK * 4)])

            plsc.subcore_barrier()

            offb = offb_smem[ch]
            nb = offb_smem[ch + 1] - offb

            def batch_body(b, carry2):
                posv = pos_v[pl.ds((offb + b) * 16, 16)]
                gposv = jnp.where(posv > EPT - 1, jnp.broadcast_to(E, (16,)),
                                  posv + base)
                pltpu.sync_copy(src_hbm.at[gposv], srcbuf)
                pltpu.sync_copy(dst_hbm.at[gposv], dstbuf)
                srcv = srcbuf[...]
                dstv = dstbuf[...]
                m = (dstv >= lo) & (dstv < hi)
                okf = jnp.where(m, 1.0, 0.0)
                pltpu.sync_copy(xin_hbm.at[srcv], gbuf)
                local = dstv - lo
                local = jnp.where(local < 0, zi, local)
                local = jnp.where(local > CHUNK - 1, zi, local)
                pclamp = jnp.where(posv > EPT - 1, zi, posv)
                for j in range(4):
                    pat = lane14 + 4 * j
                    lrep = _vg(local, pat)
                    prep = _vg(pclamp, pat)
                    sidx[pl.ds(j * 16, 16)] = lrep * 4 + hpat
                    eaidx[pl.ds(j * 16, 16)] = prep * 4 + hpat + wid * EPT4
                if SEGS == 1:
                    for j in range(4):
                        pat = lane14 + 4 * j
                        lrep = _vg(local, pat)
                        sidxs[0][pl.ds(j * 16, 16)] = lrep * 4 + hpat
                else:
                    for e in range(16):
                        le = _vg(local, jnp.broadcast_to(e, (16,)))
                        sidxs[e // EPB][pl.ds((e % EPB) * 16, 16)] = \
                            le * (4 * SEGS) + iota
                pltpu.sync_copy(eaout_hbm.at[eaidx], ea_tmp)
                for j in range(4):
                    facv = _vg(okf, lane14 + 4 * j)
                    easc[pl.ds(j * 16, 16)] = ea_tmp[pl.ds(j * 16, 16)] * facv
                for j in range(4):
                    eav = ea_tmp[pl.ds(j * 16, 16)]
                    for e2 in range(4):
                        e = j * 4 + e2
                        fac = _vg(okf, jnp.broadcast_to(e, (16,)))
                        sp = [_vg(eav, jnp.broadcast_to(e2 * 4 + h, (16,)))
                              * fac for h in range(4)]
                        bi = e // EPB
                        er = e % EPB
                        for c in range(D // 16):
                            g = gbuf.at[e][pl.ds(c * 16, 16)]
                            seg = c // 8
                            off = (c % 8) * 16
                            for h in range(4):
                                sbufs[bi].at[er * 4 * SEGS + h * SEGS + seg][
                                    pl.ds(off, 16)] = sp[h] * g
                for bi in range(NSC):
                    pltpu.sync_copy(sbufs[bi], agg_sh.at[sidxs[bi]], add=True)
                pltpu.sync_copy(easc, den_sh.at[sidx], add=True)
                return carry2

            lax.fori_loop(0, nb, batch_body, 0)

            plsc.subcore_barrier()
            pltpu.sync_copy(agg_sh.at[pl.ds(sid * RZ, RZ), :],
                            out_hbm.at[cc, ch, pl.ds(sid * RZ, RZ), :])

            @pl.when(sid == 0)
            def _():
                pltpu.sync_copy(den_sh.at[pl.ds(0, CHUNK * 4)],
                                den_hbm.at[cc, ch, pl.ds(0, CHUNK * 4)])

            return carry

        lax.fori_loop(0, NCH, chunk_body, 0)

    return sc_edge, CHUNK, NCH, SEGS, ROWS, RZ


_sc_edge_l1 = _make_sc_edge(HID)
_sc_edge_l2 = _make_sc_edge(D1)


def _run_layer_sc(maker, srcp, dstp, dst3, aefp, asrcf, adstf, xin, D):
    sc_edge, CHUNK, NCH, SEGS, ROWS, RZ = maker
    counts = _counts(dst3, CHUNK).reshape(32, OP)
    blocks = (counts + 15) >> 4
    offb = jnp.concatenate(
        [jnp.zeros((32, 1), jnp.int32), jnp.cumsum(blocks, axis=1)[:, :OP - 1]],
        axis=1).astype(jnp.int32).reshape(32 * OP)
    zeros = jnp.zeros((RZ, 128), jnp.float32)
    zerod = jnp.zeros((CHUNK * 4,), jnp.float32)
    agg, _, den = sc_edge(srcp, dstp, aefp, asrcf, adstf, xin, zeros, zerod, offb)
    feat = agg.reshape(2, NCH * CHUNK, HEADS * SEGS * 128)[:, :N]
    den = den.reshape(2, NCH * CHUNK, HEADS)[:, :N]
    return feat, den


def kernel(x, edge_index, node_type, edge_attr, edge_type, W1, att_src1, att_dst1,
           Wedge1, ntype1, etype1, Wres1, b1, gamma, beta, W2, att_src2, att_dst2,
           Wedge2, ntype2, etype2, b2):
    f32 = jnp.float32
    W1r = W1.reshape(HID, HEADS, HID)
    vsd1 = jnp.concatenate([
        jnp.einsum("khc,hc->kh", W1r, att_src1),
        jnp.einsum("khc,hc->kh", W1r, att_dst1)], axis=1)
    W2r = W2.reshape(D1, HEADS, D1)
    vsd2 = jnp.concatenate([
        jnp.einsum("khc,hc->kh", W2r, att_src2),
        jnp.einsum("khc,hc->kh", W2r, att_dst2)], axis=1)
    bd1 = jnp.zeros((D1, D1), f32)
    for h in range(HEADS):
        bd1 = bd1.at[h * HID:(h + 1) * HID, h * HID:(h + 1) * HID].set(
            W1[:, h * HID:(h + 1) * HID])
    wm2 = W2r.transpose(1, 0, 2).reshape(HEADS * D1, D1) / HEADS
    etab = jnp.concatenate([etype1 @ Wedge1, etype2 @ Wedge2], axis=1)
    we12 = jnp.concatenate([Wedge1, Wedge2], axis=1)

    nt2d = node_type.reshape(N, 1)
    et2d = edge_type.reshape(E, 1)
    src = edge_index[0]
    dst = edge_index[1]
    srcp = jnp.concatenate([src, jnp.zeros((16,), jnp.int32)])
    dstp = jnp.concatenate([dst, jnp.full((16,), N, jnp.int32)])
    dst3 = dst.reshape(32, 1, EPT)

    xin1, asd1, res1 = _node_pre(x, nt2d, ntype1, vsd1, Wres1)
    ae = _edge_pre(edge_attr, et2d, etab, we12)
    ae1f = jnp.concatenate([ae[:, :4].reshape(E * 4), jnp.zeros((64,), f32)])
    ae2f = jnp.concatenate([ae[:, 4:].reshape(E * 4), jnp.zeros((64,), f32)])
    asrc1f = asd1[:, :4].reshape(N * 4)
    adst1f = asd1[:, 4:].reshape(N * 4)

    feat1, den1 = _run_layer_sc(_sc_edge_l1, srcp, dstp, dst3, ae1f,
                                asrc1f, adst1f, xin1, HID)

    out1, stats = _combine1(feat1[0], feat1[1], den1[0], den1[1], bd1, res1,
                            b1.reshape(1, D1))
    mu = stats[0] / N
    var = stats[1] / N - mu * mu
    s = gamma / jnp.sqrt(var + 1e-5)
    t = beta - mu * s
    st = jnp.stack([s, t], axis=0)

    bnout, xin2, asd2 = _node_pre2(out1, st, nt2d, ntype2, vsd2)
    asrc2f = asd2[:, :4].reshape(N * 4)
    adst2f = asd2[:, 4:].reshape(N * 4)

    feat2, den2 = _run_layer_sc(_sc_edge_l2, srcp, dstp, dst3, ae2f,
                                asrc2f, adst2f, xin2, D1)

    return _combine2(feat2[0], feat2[1], den2[0], den2[1], wm2, bnout,
                     b2.reshape(1, D1))


# speedup vs baseline: 9.4633x; 1.0958x over previous
import functools

import jax
import jax.numpy as jnp
from jax import lax
from jax.experimental import pallas as pl
from jax.experimental.pallas import tpu as pltpu
from jax.experimental.pallas import tpu_sc as plsc

N = 10000
E = 160000
HID = 128
HEADS = 4
D1 = 512
NT = 4
ET = 8
EPT = E // 32
EPT4 = EPT * 4
NG = EPT // 16
GEOM = {128: dict(CHUNK=1024, NCH=10), 512: dict(CHUNK=256, NCH=40)}
PREG = EPT + 16 * 48 + 40
OP = 48


def _node_pre_kernel(x_ref, nt_ref, ntab_ref, vsd_ref, wres_ref, xin_ref, asd_ref, res_ref):
    x = x_ref[...]
    nt = nt_ref[...]
    emb = jnp.zeros_like(x)
    for t in range(NT):
        emb = jnp.where(nt == t, jnp.broadcast_to(ntab_ref[t:t + 1, :], x.shape), emb)
    xin = x + emb
    xin_ref[...] = xin
    asd_ref[...] = jnp.dot(xin, vsd_ref[...], preferred_element_type=jnp.float32)
    res_ref[...] = jnp.dot(x, wres_ref[...], preferred_element_type=jnp.float32)


def _node_pre(x, nt2d, ntab, vsd, wres):
    n, d = x.shape
    blk = 1000
    return pl.pallas_call(
        _node_pre_kernel,
        grid=(n // blk,),
        in_specs=[
            pl.BlockSpec((blk, d), lambda i: (i, 0)),
            pl.BlockSpec((blk, 1), lambda i: (i, 0)),
            pl.BlockSpec(ntab.shape, lambda i: (0, 0)),
            pl.BlockSpec(vsd.shape, lambda i: (0, 0)),
            pl.BlockSpec(wres.shape, lambda i: (0, 0)),
        ],
        out_specs=[
            pl.BlockSpec((blk, d), lambda i: (i, 0)),
            pl.BlockSpec((blk, 8), lambda i: (i, 0)),
            pl.BlockSpec((blk, wres.shape[1]), lambda i: (i, 0)),
        ],
        out_shape=[
            jax.ShapeDtypeStruct((n, d), jnp.float32),
            jax.ShapeDtypeStruct((n, 8), jnp.float32),
            jax.ShapeDtypeStruct((n, wres.shape[1]), jnp.float32),
        ],
    )(x, nt2d, ntab, vsd, wres)


def _node_pre2_kernel(h_ref, st_ref, nt_ref, ntab_ref, vsd_ref, bn_ref, xin_ref, asd_ref):
    h = h_ref[...]
    nt = nt_ref[...]
    s = st_ref[0:1, :]
    t = st_ref[1:2, :]
    bn = h * s + t
    emb = jnp.zeros_like(h)
    for tt in range(NT):
        emb = jnp.where(nt == tt, jnp.broadcast_to(ntab_ref[tt:tt + 1, :], h.shape), emb)
    xin = bn + emb
    bn_ref[...] = bn
    xin_ref[...] = xin
    asd_ref[...] = jnp.dot(xin, vsd_ref[...], preferred_element_type=jnp.float32)


def _node_pre2(h, st, nt2d, ntab, vsd):
    n, d = h.shape
    blk = 1000
    return pl.pallas_call(
        _node_pre2_kernel,
        grid=(n // blk,),
        in_specs=[
            pl.BlockSpec((blk, d), lambda i: (i, 0)),
            pl.BlockSpec(st.shape, lambda i: (0, 0)),
            pl.BlockSpec((blk, 1), lambda i: (i, 0)),
            pl.BlockSpec(ntab.shape, lambda i: (0, 0)),
            pl.BlockSpec(vsd.shape, lambda i: (0, 0)),
        ],
        out_specs=[
            pl.BlockSpec((blk, d), lambda i: (i, 0)),
            pl.BlockSpec((blk, d), lambda i: (i, 0)),
            pl.BlockSpec((blk, 8), lambda i: (i, 0)),
        ],
        out_shape=[
            jax.ShapeDtypeStruct((n, d), jnp.float32),
            jax.ShapeDtypeStruct((n, d), jnp.float32),
            jax.ShapeDtypeStruct((n, 8), jnp.float32),
        ],
    )(h, st, nt2d, ntab, vsd)


def _edge_pre_kernel(ea_ref, et_ref, etab_ref, we_ref, ae_ref):
    et = et_ref[...]
    acc = jnp.zeros((ea_ref.shape[0], 8), jnp.float32)
    for t in range(ET):
        acc = jnp.where(et == t, jnp.broadcast_to(etab_ref[t:t + 1, :], acc.shape), acc)
    ae_ref[...] = jnp.dot(ea_ref[...], we_ref[...], preferred_element_type=jnp.float32) + acc


def _edge_pre(edge_attr, et2d, etab, we):
    e, d = edge_attr.shape
    blk = 2000
    return pl.pallas_call(
        _edge_pre_kernel,
        grid=(e // blk,),
        in_specs=[
            pl.BlockSpec((blk, d), lambda i: (i, 0)),
            pl.BlockSpec((blk, 1), lambda i: (i, 0)),
            pl.BlockSpec(etab.shape, lambda i: (0, 0)),
            pl.BlockSpec(we.shape, lambda i: (0, 0)),
        ],
        out_specs=pl.BlockSpec((blk, 8), lambda i: (i, 0)),
        out_shape=jax.ShapeDtypeStruct((e, 8), jnp.float32),
    )(edge_attr, et2d, etab, we)


def _counts_kernel(csh, dst_ref, cnt_ref):
    cid = dst_ref[...] >> csh
    cidr = cid.reshape(1, EPT, 1)
    ids = lax.broadcasted_iota(jnp.int32, (1, 1, OP), 2)
    eq = jnp.where(cidr == ids, 1, 0)
    cnt_ref[...] = jnp.sum(eq, axis=1, keepdims=True)


def _counts(dst3, chunk):
    csh = chunk.bit_length() - 1
    return pl.pallas_call(
        functools.partial(_counts_kernel, csh),
        grid=(32,),
        in_specs=[pl.BlockSpec((1, 1, EPT), lambda i: (i, 0, 0))],
        out_specs=pl.BlockSpec((1, 1, OP), lambda i: (i, 0, 0)),
        out_shape=jax.ShapeDtypeStruct((32, 1, OP), jnp.int32),
    )(dst3)


def _combine1_kernel(aggA_ref, aggB_ref, denA_ref, denB_ref, bd_ref, res_ref, b_ref,
                     out_ref, stats_ref):
    i = pl.program_id(0)
    agg = aggA_ref[...] + aggB_ref[...]
    den = denA_ref[...] + denB_ref[...] + 1e-16
    blk = agg.shape[0]
    d = agg.shape[1] // HEADS
    deni = (1.0 / den).reshape(blk, HEADS, 1)
    norm = (agg.reshape(blk, HEADS, d) * deni).reshape(blk, HEADS * d)
    out = jnp.dot(norm, bd_ref[...], preferred_element_type=jnp.float32)
    out = out + res_ref[...] + b_ref[...]
    out_ref[...] = out
    s = jnp.sum(out, axis=0, keepdims=True)
    ss = jnp.sum(out * out, axis=0, keepdims=True)
    st = jnp.concatenate([s, ss, jnp.zeros((6, out.shape[1]), jnp.float32)], axis=0)

    @pl.when(i == 0)
    def _():
        stats_ref[...] = jnp.zeros_like(stats_ref)

    stats_ref[...] += st


def _combine1(aggA, aggB, denA, denB, bd, res, b2d):
    n, dcat = aggA.shape
    blk = 1000
    return pl.pallas_call(
        _combine1_kernel,
        grid=(n // blk,),
        in_specs=[
            pl.BlockSpec((blk, dcat), lambda i: (i, 0)),
            pl.BlockSpec((blk, dcat), lambda i: (i, 0)),
            pl.BlockSpec((blk, HEADS), lambda i: (i, 0)),
            pl.BlockSpec((blk, HEADS), lambda i: (i, 0)),
            pl.BlockSpec(bd.shape, lambda i: (0, 0)),
            pl.BlockSpec((blk, bd.shape[1]), lambda i: (i, 0)),
            pl.BlockSpec(b2d.shape, lambda i: (0, 0)),
        ],
        out_specs=[
            pl.BlockSpec((blk, bd.shape[1]), lambda i: (i, 0)),
            pl.BlockSpec((8, bd.shape[1]), lambda i: (0, 0)),
        ],
        out_shape=[
            jax.ShapeDtypeStruct((n, bd.shape[1]), jnp.float32),
            jax.ShapeDtypeStruct((8, bd.shape[1]), jnp.float32),
        ],
    )(aggA, aggB, denA, denB, bd, res, b2d)


def _combine2_kernel(aggA_ref, aggB_ref, denA_ref, denB_ref, wm_ref, res_ref, b_ref, out_ref):
    agg = aggA_ref[...] + aggB_ref[...]
    den = denA_ref[...] + denB_ref[...] + 1e-16
    blk = agg.shape[0]
    d = agg.shape[1] // HEADS
    deni = (1.0 / den).reshape(blk, HEADS, 1)
    norm = (agg.reshape(blk, HEADS, d) * deni).reshape(blk, HEADS * d)
    out = jnp.dot(norm, wm_ref[...], preferred_element_type=jnp.float32)
    out_ref[...] = out + res_ref[...] + b_ref[...]


def _combine2(aggA, aggB, denA, denB, wm, res, b2d):
    n, dcat = aggA.shape
    blk = 1000
    return pl.pallas_call(
        _combine2_kernel,
        grid=(n // blk,),
        in_specs=[
            pl.BlockSpec((blk, dcat), lambda i: (i, 0)),
            pl.BlockSpec((blk, dcat), lambda i: (i, 0)),
            pl.BlockSpec((blk, HEADS), lambda i: (i, 0)),
            pl.BlockSpec((blk, HEADS), lambda i: (i, 0)),
            pl.BlockSpec(wm.shape, lambda i: (0, 0)),
            pl.BlockSpec((blk, wm.shape[1]), lambda i: (i, 0)),
            pl.BlockSpec(b2d.shape, lambda i: (0, 0)),
        ],
        out_specs=pl.BlockSpec((blk, wm.shape[1]), lambda i: (i, 0)),
        out_shape=jax.ShapeDtypeStruct((n, wm.shape[1]), jnp.float32),
    )(aggA, aggB, denA, denB, wm, res, b2d)


def _vg(v, idx):
    dn = lax.GatherDimensionNumbers(
        offset_dims=(), collapsed_slice_dims=(0,), start_index_map=(0,))
    return lax.gather(v, idx[:, None], dn, (1,),
                      mode=lax.GatherScatterMode.PROMISE_IN_BOUNDS)


def _make_sc_edge(D):
    geom = GEOM[D]
    CHUNK, NCH = geom["CHUNK"], geom["NCH"]
    CSH = CHUNK.bit_length() - 1
    NCV = (NCH + 15) // 16
    SEGS = D // 128
    NSC = (64 * SEGS + 127) // 128
    RPB = 64 * SEGS // NSC
    EPB = 16 // NSC
    ROWS = CHUNK * 4 * SEGS
    RZ = ROWS // 16

    mesh = plsc.VectorSubcoreMesh(core_axis_name="c", subcore_axis_name="s")

    @functools.partial(
        pl.kernel,
        mesh=mesh,
        out_type=[jax.ShapeDtypeStruct((2, NCH, ROWS, 128), jnp.float32),
                  jax.ShapeDtypeStruct((32 * EPT4,), jnp.float32),
                  jax.ShapeDtypeStruct((2, NCH, CHUNK * 4), jnp.float32)],
        scratch_types=[
            pltpu.VMEM((EPT + 64,), jnp.int32),
            pltpu.VMEM((EPT + 64,), jnp.int32),
            pltpu.VMEM((EPT4 + 64,), jnp.float32),
            pltpu.VMEM((PREG,), jnp.int32),
            pltpu.VMEM((PREG,), jnp.int32),
            pltpu.VMEM((16, D), jnp.float32),
            pltpu.VMEM((64,), jnp.float32),
            pltpu.VMEM((64,), jnp.int32),
        ] + [pltpu.VMEM((RPB, 128), jnp.float32) for _ in range(NSC)]
          + [pltpu.VMEM((RPB,), jnp.int32) for _ in range(NSC)] + [
            pltpu.VMEM((64,), jnp.int32),
            pltpu.VMEM((64,), jnp.float32),
            pltpu.VMEM((64,), jnp.float32),
            pltpu.VMEM((64,), jnp.float32),
            pltpu.VMEM((64,), jnp.int32),
            pltpu.VMEM((64,), jnp.int32),
            pltpu.VMEM((64,), jnp.int32),
            pltpu.VMEM((64,), jnp.int32),
            pltpu.VMEM((OP,), jnp.int32),
            pltpu.SMEM((OP,), jnp.int32),
            pltpu.VMEM_SHARED((32 * OP,), jnp.int32),
            pltpu.VMEM_SHARED((16 * PREG,), jnp.int32),
            pltpu.VMEM_SHARED((16 * PREG,), jnp.int32),
            pltpu.SemaphoreType.DMA,
            pltpu.SemaphoreType.DMA,
            pltpu.VMEM_SHARED((ROWS, 128), jnp.float32),
            pltpu.VMEM_SHARED((CHUNK * 4 + 16,), jnp.float32),
        ],
    )
    def sc_edge(src_hbm, dst_hbm, aef_hbm, asrcf_hbm, adstf_hbm, xin_hbm,
                zeros_hbm, zerod_hbm, offb_hbm,
                out_hbm, eaout_hbm, den_hbm,
                src_v, dst_v, ea_v, pos_v, csd_v, gbuf, easc, sidx,
                *rest):
        sbufs = rest[:NSC]
        sidxs = rest[NSC:2 * NSC]
        (eaidx, ea_tmp, atmp, btmp, cposbuf, cvalbuf, csbuf, cdbuf,
         offb_v, offb_smem, offs_sh, pos_sp, csd_sp, sem1, sem2,
         agg_sh, den_sh) = rest[2 * NSC:]
        cc = lax.axis_index("c")
        sid = lax.axis_index("s")
        wid = sid * 2 + cc
        iota = lax.iota(jnp.int32, 16)
        lane14 = iota >> 2
        hpat = iota & 3
        zi = jnp.zeros((16,), jnp.int32)
        zf = jnp.zeros((16,), jnp.float32)
        padpos = jnp.broadcast_to(EPT, (16,))
        nvec = jnp.broadcast_to(N, (16,))

        base = wid * EPT
        pltpu.sync_copy(src_hbm.at[pl.ds(base, EPT)], src_v.at[pl.ds(0, EPT)])
        pltpu.sync_copy(dst_hbm.at[pl.ds(base, EPT)], dst_v.at[pl.ds(0, EPT)])
        pltpu.sync_copy(aef_hbm.at[pl.ds(base * 4, EPT4)],
                        ea_v.at[pl.ds(0, EPT4)])
        for q in range(4):
            src_v[pl.ds(EPT + q * 16, 16)] = zi
            dst_v[pl.ds(EPT + q * 16, 16)] = nvec

        @pl.when(sid == 0)
        def _():
            pltpu.sync_copy(offb_hbm, offs_sh)

        plsc.subcore_barrier()
        pltpu.sync_copy(offs_sh.at[pl.ds(wid * OP, OP)], offb_v)
        pltpu.sync_copy(offs_sh.at[pl.ds(wid * OP, OP)], offb_smem)

        def ea_body(i, carry):
            srcv = src_v[pl.ds(i * 16, 16)]
            dstv = dst_v[pl.ds(i * 16, 16)]
            for j in range(4):
                pat = lane14 + 4 * j
                srep = _vg(srcv, pat)
                drep = _vg(dstv, pat)
                drep = jnp.where(drep > N - 1, zi, drep)
                eaidx[pl.ds(j * 16, 16)] = srep * 4 + hpat
                sidx[pl.ds(j * 16, 16)] = drep * 4 + hpat
            pltpu.sync_copy(asrcf_hbm.at[eaidx], atmp)
            pltpu.sync_copy(adstf_hbm.at[sidx], btmp)
            for j in range(4):
                v = atmp[pl.ds(j * 16, 16)] + btmp[pl.ds(j * 16, 16)] \
                    + ea_v[pl.ds(i * 64 + j * 16, 16)]
                v = jnp.where(v >= 0.0, v, 0.2 * v)
                ea_v[pl.ds(i * 64 + j * 16, 16)] = jnp.exp(v)
            return carry

        lax.fori_loop(0, NG + 1, ea_body, 0)
        for q in range(4):
            ea_v[pl.ds(EPT4 + q * 16, 16)] = zf
        pltpu.sync_copy(ea_v.at[pl.ds(0, EPT4)],
                        eaout_hbm.at[pl.ds(wid * EPT4, EPT4)])

        def pf_body(i, carry):
            pos_v[pl.ds(i * 16, 16)] = padpos
            csd_v[pl.ds(i * 16, 16)] = nvec * 16384
            return carry

        lax.fori_loop(0, PREG // 16, pf_body, 0)
        pltpu.sync_copy(pos_v, pos_sp.at[pl.ds(sid * PREG, PREG)])
        pltpu.sync_copy(csd_v, csd_sp.at[pl.ds(sid * PREG, PREG)])

        def comp_body(g, counters):
            cnts = list(counters)
            for j4 in range(4):
                i = g * 4 + j4
                dstv = dst_v[pl.ds(i * 16, 16)]
                posv = i * 16 + iota
                okm = posv < EPT
                cid = jnp.where(okm, dstv >> CSH, iota * 0 + 99)
                rank = zi
                hists = [zi] * NCV
                for l in range(16):
                    cl = _vg(cid, jnp.broadcast_to(l, (16,)))
                    rank = rank + jnp.where((cid == cl) & (iota > l), 1, 0)
                    for k in range(NCV):
                        hists[k] = hists[k] + jnp.where(cl - 16 * k == iota, 1, 0)
                slot = zi
                for k in range(NCV):
                    ck = cid - 16 * k
                    ck = jnp.where(ck < 0, zi, ck)
                    ck = jnp.where(ck > 15, zi, ck)
                    bask = _vg(cnts[k] + offb_v[pl.ds(16 * k, 16)] * 16, ck)
                    if k == 0:
                        slot = bask
                    else:
                        slot = jnp.where(cid > 16 * k - 1, bask, slot)
                slot = slot + rank
                slot = jnp.where(okm, slot, PREG - 16 + (iota & 7))
                cposbuf[pl.ds(j4 * 16, 16)] = slot + sid * PREG
                cvalbuf[pl.ds(j4 * 16, 16)] = posv
                csbuf[pl.ds(j4 * 16, 16)] = \
                    src_v[pl.ds(i * 16, 16)] + dstv * 16384
                for k in range(NCV):
                    cnts[k] = cnts[k] + hists[k]
            pltpu.sync_copy(cvalbuf, pos_sp.at[cposbuf])
            pltpu.sync_copy(csbuf, csd_sp.at[cposbuf])
            return tuple(cnts)

        lax.fori_loop(0, (NG + 1 + 3) // 4, comp_body, tuple([zi] * NCV))
        pltpu.sync_copy(pos_sp.at[pl.ds(sid * PREG, PREG)], pos_v)
        pltpu.sync_copy(csd_sp.at[pl.ds(sid * PREG, PREG)], csd_v)

        def chunk_body(ch, carry):
            lo = ch * CHUNK
            hi = lo + CHUNK
            pltpu.sync_copy(zeros_hbm, agg_sh.at[pl.ds(sid * RZ, RZ), :])

            @pl.when(sid == 0)
            def _():
                pltpu.sync_copy(zerod_hbm, den_sh.at[pl.ds(0, CHUNK * 4)])

            plsc.subcore_barrier()

            offb = offb_smem[ch]
            nb = offb_smem[ch + 1] - offb

            def batch_body(b, carry2):
                posv = pos_v[pl.ds((offb + b) * 16, 16)]
                sd = csd_v[pl.ds((offb + b) * 16, 16)]
                srcv = sd & 16383
                dstv = sd >> 14
                m = (dstv >= lo) & (dstv < hi)
                okf = jnp.where(m, 1.0, 0.0)
                dma_x = pltpu.async_copy(xin_hbm.at[srcv], gbuf, sem2)
                local = dstv - lo
                local = jnp.where(local < 0, zi, local)
                local = jnp.where(local > CHUNK - 1, zi, local)
                pclamp = jnp.where(posv > EPT - 1, zi, posv)
                for j in range(4):
                    pat = lane14 + 4 * j
                    lrep = _vg(local, pat)
                    prep = _vg(pclamp, pat)
                    sidx[pl.ds(j * 16, 16)] = lrep * 4 + hpat
                    eaidx[pl.ds(j * 16, 16)] = prep * 4 + hpat + wid * EPT4
                dma_e = pltpu.async_copy(eaout_hbm.at[eaidx], ea_tmp, sem1)
                if SEGS == 1:
                    for j in range(4):
                        pat = lane14 + 4 * j
                        lrep = _vg(local, pat)
                        sidxs[0][pl.ds(j * 16, 16)] = lrep * 4 + hpat
                else:
                    for e in range(16):
                        le = _vg(local, jnp.broadcast_to(e, (16,)))
                        sidxs[e // EPB][pl.ds((e % EPB) * 16, 16)] = \
                            le * (4 * SEGS) + iota
                dma_e.wait()
                dma_x.wait()
                for j in range(4):
                    facv = _vg(okf, lane14 + 4 * j)
                    easc[pl.ds(j * 16, 16)] = ea_tmp[pl.ds(j * 16, 16)] * facv
                for j in range(4):
                    eav = ea_tmp[pl.ds(j * 16, 16)]
                    for e2 in range(4):
                        e = j * 4 + e2
                        fac = _vg(okf, jnp.broadcast_to(e, (16,)))
                        sp = [_vg(eav, jnp.broadcast_to(e2 * 4 + h, (16,)))
                              * fac for h in range(4)]
                        bi = e // EPB
                        er = e % EPB
                        for c in range(D // 16):
                            g = gbuf.at[e][pl.ds(c * 16, 16)]
                            seg = c // 8
                            off = (c % 8) * 16
                            for h in range(4):
                                sbufs[bi].at[er * 4 * SEGS + h * SEGS + seg][
                                    pl.ds(off, 16)] = sp[h] * g
                for bi in range(NSC):
                    pltpu.sync_copy(sbufs[bi], agg_sh.at[sidxs[bi]], add=True)
                pltpu.sync_copy(easc, den_sh.at[sidx], add=True)
                return carry2

            lax.fori_loop(0, nb, batch_body, 0)

            plsc.subcore_barrier()
            pltpu.sync_copy(agg_sh.at[pl.ds(sid * RZ, RZ), :],
                            out_hbm.at[cc, ch, pl.ds(sid * RZ, RZ), :])

            @pl.when(sid == 0)
            def _():
                pltpu.sync_copy(den_sh.at[pl.ds(0, CHUNK * 4)],
                                den_hbm.at[cc, ch, pl.ds(0, CHUNK * 4)])

            return carry

        lax.fori_loop(0, NCH, chunk_body, 0)

    return sc_edge, CHUNK, NCH, SEGS, ROWS, RZ


_sc_edge_l1 = _make_sc_edge(HID)
_sc_edge_l2 = _make_sc_edge(D1)


def _run_layer_sc(maker, srcp, dstp, dst3, aefp, asrcf, adstf, xin, D):
    sc_edge, CHUNK, NCH, SEGS, ROWS, RZ = maker
    counts = _counts(dst3, CHUNK).reshape(32, OP)
    blocks = (counts + 15) >> 4
    offb = jnp.concatenate(
        [jnp.zeros((32, 1), jnp.int32), jnp.cumsum(blocks, axis=1)[:, :OP - 1]],
        axis=1).astype(jnp.int32).reshape(32 * OP)
    zeros = jnp.zeros((RZ, 128), jnp.float32)
    zerod = jnp.zeros((CHUNK * 4,), jnp.float32)
    agg, _, den = sc_edge(srcp, dstp, aefp, asrcf, adstf, xin, zeros, zerod, offb)
    feat = agg.reshape(2, NCH * CHUNK, HEADS * SEGS * 128)[:, :N]
    den = den.reshape(2, NCH * CHUNK, HEADS)[:, :N]
    return feat, den


def kernel(x, edge_index, node_type, edge_attr, edge_type, W1, att_src1, att_dst1,
           Wedge1, ntype1, etype1, Wres1, b1, gamma, beta, W2, att_src2, att_dst2,
           Wedge2, ntype2, etype2, b2):
    f32 = jnp.float32
    W1r = W1.reshape(HID, HEADS, HID)
    vsd1 = jnp.concatenate([
        jnp.einsum("khc,hc->kh", W1r, att_src1),
        jnp.einsum("khc,hc->kh", W1r, att_dst1)], axis=1)
    W2r = W2.reshape(D1, HEADS, D1)
    vsd2 = jnp.concatenate([
        jnp.einsum("khc,hc->kh", W2r, att_src2),
        jnp.einsum("khc,hc->kh", W2r, att_dst2)], axis=1)
    bd1 = jnp.zeros((D1, D1), f32)
    for h in range(HEADS):
        bd1 = bd1.at[h * HID:(h + 1) * HID, h * HID:(h + 1) * HID].set(
            W1[:, h * HID:(h + 1) * HID])
    wm2 = W2r.transpose(1, 0, 2).reshape(HEADS * D1, D1) / HEADS
    etab = jnp.concatenate([etype1 @ Wedge1, etype2 @ Wedge2], axis=1)
    we12 = jnp.concatenate([Wedge1, Wedge2], axis=1)

    nt2d = node_type.reshape(N, 1)
    et2d = edge_type.reshape(E, 1)
    src = edge_index[0]
    dst = edge_index[1]
    srcp = jnp.concatenate([src, jnp.zeros((16,), jnp.int32)])
    dstp = jnp.concatenate([dst, jnp.full((16,), N, jnp.int32)])
    dst3 = dst.reshape(32, 1, EPT)

    xin1, asd1, res1 = _node_pre(x, nt2d, ntype1, vsd1, Wres1)
    ae = _edge_pre(edge_attr, et2d, etab, we12)
    ae1f = jnp.concatenate([ae[:, :4].reshape(E * 4), jnp.zeros((64,), f32)])
    ae2f = jnp.concatenate([ae[:, 4:].reshape(E * 4), jnp.zeros((64,), f32)])
    asrc1f = asd1[:, :4].reshape(N * 4)
    adst1f = asd1[:, 4:].reshape(N * 4)

    feat1, den1 = _run_layer_sc(_sc_edge_l1, srcp, dstp, dst3, ae1f,
                                asrc1f, adst1f, xin1, HID)

    out1, stats = _combine1(feat1[0], feat1[1], den1[0], den1[1], bd1, res1,
                            b1.reshape(1, D1))
    mu = stats[0] / N
    var = stats[1] / N - mu * mu
    s = gamma / jnp.sqrt(var + 1e-5)
    t = beta - mu * s
    st = jnp.stack([s, t], axis=0)

    bnout, xin2, asd2 = _node_pre2(out1, st, nt2d, ntype2, vsd2)
    asrc2f = asd2[:, :4].reshape(N * 4)
    adst2f = asd2[:, 4:].reshape(N * 4)

    feat2, den2 = _run_layer_sc(_sc_edge_l2, srcp, dstp, dst3, ae2f,
                                asrc2f, adst2f, xin2, D1)

    return _combine2(feat2[0], feat2[1], den2[0], den2[1], wm2, bnout,
                     b2.reshape(1, D1))
